# baseline (speedup 1.0000x reference)
import functools

import jax
import jax.numpy as jnp
from jax import lax
from jax.experimental import pallas as pl
from jax.experimental.pallas import tpu as pltpu
from jax.experimental.pallas import tpu_sc as plsc

B = 4096
L = 50
D = 64
NW = 32
BW = B // NW
CH = 128
NCH = (BW * L) // CH
NCHP = 56
MAGIC = 41944


def _body(user_tab, item_tab, uid_hbm, iid_hbm, hist_hbm, histn_hbm, out_hbm,
          idx_v, uidx_v, iidx_v, rows_v, acc_v, urows_v, irows_v, histn_v,
          out_buf, recip_v,
          sem_g0, sem_g1, sem_u, sem_i, sem_n):
    c = lax.axis_index("c")
    s = lax.axis_index("s")
    wid = s * 2 + c
    base = wid * BW

    pltpu.sync_copy(hist_hbm.at[wid], idx_v)

    pltpu.make_async_copy(uid_hbm.at[pl.ds(base, BW)], uidx_v, sem_u).start()
    pltpu.make_async_copy(iid_hbm.at[pl.ds(base, BW)], iidx_v, sem_i).start()
    pltpu.make_async_copy(
        histn_hbm.at[pl.ds(base, BW), :], histn_v, sem_n).start()

    pltpu.make_async_copy(item_tab.at[idx_v.at[0]], rows_v.at[0], sem_g0).start()
    pltpu.make_async_copy(item_tab.at[idx_v.at[1]], rows_v.at[1], sem_g1).start()

    pltpu.make_async_copy(uid_hbm.at[pl.ds(base, BW)], uidx_v, sem_u).wait()
    pltpu.make_async_copy(user_tab.at[uidx_v], urows_v, sem_u).start()
    pltpu.make_async_copy(iid_hbm.at[pl.ds(base, BW)], iidx_v, sem_i).wait()
    pltpu.make_async_copy(item_tab.at[iidx_v], irows_v, sem_i).start()

    zero = jnp.zeros((16,), jnp.float32)

    def zero_body(i, carry):
        for j in range(4):
            acc_v[i, pl.ds(j * 16, 16)] = zero
        return carry

    lax.fori_loop(0, BW, zero_body, 0)

    pltpu.make_async_copy(
        histn_hbm.at[pl.ds(base, BW), :], histn_v, sem_n).wait()

    def cnt_body(b, carry):
        cnt = jnp.zeros((16,), jnp.int32)
        for k in range(4):
            x = histn_v[b, pl.ds(k * 16, 16)]
            cnt = cnt + plsc.all_reduce_population_count(x != 0)
        recip_v[b, :] = 1.0 / (cnt.astype(jnp.float32) + 1e-9)
        return carry

    lax.fori_loop(0, BW, cnt_body, 0)

    def main_body(it, carry):
        for p in range(2):
            cc = it * 2 + p
            sem = sem_g0 if p == 0 else sem_g1
            pltpu.make_async_copy(
                item_tab.at[idx_v.at[cc]], rows_v.at[p], sem).wait()
            for i in range(CH):
                dst = lax.shift_right_logical((cc * CH + i) * MAGIC, 21)
                for j in range(4):
                    v = rows_v[p, i, pl.ds(j * 16, 16)]
                    plsc.addupdate(acc_v.at[dst, pl.ds(j * 16, 16)], v)

            @pl.when(cc + 2 < NCH)
            def _():
                pltpu.make_async_copy(
                    item_tab.at[idx_v.at[cc + 2]], rows_v.at[p], sem).start()
        return carry

    lax.fori_loop(0, NCH // 2, main_body, 0)

    pltpu.make_async_copy(user_tab.at[uidx_v], urows_v, sem_u).wait()
    pltpu.make_async_copy(item_tab.at[iidx_v], irows_v, sem_i).wait()

    def asm_body(b, carry):
        rec = recip_v[b, :]
        for j in range(4):
            sl = pl.ds(j * 16, 16)
            out_buf[b, pl.ds(j * 16, 16)] = urows_v[b, sl]
            out_buf[b, pl.ds(D + j * 16, 16)] = irows_v[b, sl]
            out_buf[b, pl.ds(2 * D + j * 16, 16)] = acc_v[b, sl] * rec
        return carry

    lax.fori_loop(0, BW, asm_body, 0)
    pltpu.sync_copy(out_buf, out_hbm.at[pl.ds(base, BW), :])


_sc_call = functools.partial(
    pl.kernel,
    mesh=plsc.VectorSubcoreMesh(core_axis_name="c", subcore_axis_name="s"),
    out_type=jax.ShapeDtypeStruct((B, 3 * D), jnp.float32),
    compiler_params=pltpu.CompilerParams(
        needs_layout_passes=False, use_tc_tiling_on_sc=False),
    scratch_types=[
        pltpu.VMEM((NCHP, CH), jnp.int32),
        pltpu.VMEM((BW,), jnp.int32),
        pltpu.VMEM((BW,), jnp.int32),
        pltpu.VMEM((2, CH, D), jnp.float32),
        pltpu.VMEM((BW, D), jnp.float32),
        pltpu.VMEM((BW, D), jnp.float32),
        pltpu.VMEM((BW, D), jnp.float32),
        pltpu.VMEM((BW, D), jnp.int32),
        pltpu.VMEM((BW, 3 * D), jnp.float32),
        pltpu.VMEM((BW, 16), jnp.float32),
        pltpu.SemaphoreType.DMA,
        pltpu.SemaphoreType.DMA,
        pltpu.SemaphoreType.DMA,
        pltpu.SemaphoreType.DMA,
        pltpu.SemaphoreType.DMA,
    ],
)(_body)


@jax.jit
def kernel(user_tab, item_tab, user_id, item_id, history_item_id):
    uid = user_id.astype(jnp.int32)
    iid = item_id.astype(jnp.int32)
    hid = history_item_id.astype(jnp.int32)
    hist = hid.reshape(NW, BW * L)
    hist = jnp.pad(hist, ((0, 0), (0, NCHP * CH - BW * L)))
    hist = hist.reshape(NW, NCHP, CH)
    histn = jnp.pad(hid, ((0, 0), (0, D - L)))
    return _sc_call(user_tab, item_tab, uid, iid, hist, histn)
